# initial kernel scaffold (unmeasured)
import jax
import jax.numpy as jnp
from jax import lax
from jax.experimental import pallas as pl
from jax.experimental.pallas import tpu as pltpu


def kernel(
    x,
):
    def body(*refs):
        pass

    out_shape = jax.ShapeDtypeStruct(..., jnp.float32)
    return pl.pallas_call(body, out_shape=out_shape)(...)


# baseline (device time: 78220 ns/iter reference)
import jax
import jax.numpy as jnp
from jax import lax
from jax.experimental import pallas as pl
from jax.experimental.pallas import tpu as pltpu

N_DEV = 4


def kernel(x):
    m, n = x.shape

    def body(x_ref, out_ref, comm_ref, send_sems, recv_sems):
        my_pos = lax.axis_index("i")
        left = (my_pos - 1) % N_DEV
        right = (my_pos + 1) % N_DEV

        barrier_sem = pltpu.get_barrier_semaphore()
        for nbr in [left, right]:
            pl.semaphore_signal(
                barrier_sem, inc=1,
                device_id=(nbr,), device_id_type=pl.DeviceIdType.MESH,
            )
        pl.semaphore_wait(barrier_sem, 2)

        out_ref[:, :] = x_ref[:, :]
        comm_ref[0, :, :] = x_ref[:, :]

        for h in range(N_DEV - 1):
            send_slot = h % 2
            recv_slot = (h + 1) % 2
            rdma = pltpu.make_async_remote_copy(
                src_ref=comm_ref.at[send_slot],
                dst_ref=comm_ref.at[recv_slot],
                send_sem=send_sems.at[send_slot],
                recv_sem=recv_sems.at[recv_slot],
                device_id=(right,),
                device_id_type=pl.DeviceIdType.MESH,
            )
            rdma.start()
            rdma.wait()
            out_ref[:, :] = out_ref[:, :] + comm_ref[recv_slot, :, :]

    return pl.pallas_call(
        body,
        out_shape=jax.ShapeDtypeStruct((m, n), x.dtype),
        in_specs=[pl.BlockSpec(memory_space=pltpu.VMEM)],
        out_specs=pl.BlockSpec(memory_space=pltpu.VMEM),
        scratch_shapes=[
            pltpu.VMEM((2, m, n), x.dtype),
            pltpu.SemaphoreType.DMA((2,)),
            pltpu.SemaphoreType.DMA((2,)),
        ],
        compiler_params=pltpu.CompilerParams(collective_id=0),
    )(x)


# device time: 49614 ns/iter; 1.5766x vs baseline; 1.5766x over previous
import jax
import jax.numpy as jnp
from jax import lax
from jax.experimental import pallas as pl
from jax.experimental.pallas import tpu as pltpu

N_DEV = 4


def kernel(x):
    m, n = x.shape
    m_chunk = m // N_DEV

    def body(x_ref, out_ref, comm_ref, send_sems, recv_sems):
        my_pos = lax.axis_index("i")
        left = (my_pos - 1) % N_DEV
        right = (my_pos + 1) % N_DEV

        barrier_sem = pltpu.get_barrier_semaphore()
        for nbr in [left, right]:
            pl.semaphore_signal(
                barrier_sem, inc=1,
                device_id=(nbr,), device_id_type=pl.DeviceIdType.MESH,
            )
        pl.semaphore_wait(barrier_sem, 2)

        def chunk(ref, c):
            return ref[pl.ds(c * m_chunk, m_chunk), :]

        def hop(t):
            send_slot = t % 2
            recv_slot = (t + 1) % 2
            rdma = pltpu.make_async_remote_copy(
                src_ref=comm_ref.at[send_slot],
                dst_ref=comm_ref.at[recv_slot],
                send_sem=send_sems.at[send_slot],
                recv_sem=recv_sems.at[recv_slot],
                device_id=(right,),
                device_id_type=pl.DeviceIdType.MESH,
            )
            rdma.start()
            rdma.wait()
            return recv_slot

        comm_ref[0, :, :] = chunk(x_ref, my_pos)
        for h in range(N_DEV - 1):
            recv_slot = hop(h)
            c = (my_pos - h - 1) % N_DEV
            comm_ref[recv_slot, :, :] = comm_ref[recv_slot, :, :] + chunk(x_ref, c)
        out_ref[pl.ds(((my_pos + 1) % N_DEV) * m_chunk, m_chunk), :] = (
            comm_ref[1, :, :]
        )

        for g in range(N_DEV - 1):
            recv_slot = hop(N_DEV - 1 + g)
            c = (my_pos - g) % N_DEV
            out_ref[pl.ds(c * m_chunk, m_chunk), :] = comm_ref[recv_slot, :, :]

    return pl.pallas_call(
        body,
        out_shape=jax.ShapeDtypeStruct((m, n), x.dtype),
        in_specs=[pl.BlockSpec(memory_space=pltpu.VMEM)],
        out_specs=pl.BlockSpec(memory_space=pltpu.VMEM),
        scratch_shapes=[
            pltpu.VMEM((2, m_chunk, n), x.dtype),
            pltpu.SemaphoreType.DMA((2,)),
            pltpu.SemaphoreType.DMA((2,)),
        ],
        compiler_params=pltpu.CompilerParams(collective_id=0),
    )(x)


# device time: 29404 ns/iter; 2.6602x vs baseline; 1.6873x over previous
import jax
import jax.numpy as jnp
from jax import lax
from jax.experimental import pallas as pl
from jax.experimental.pallas import tpu as pltpu

N_DEV = 4


def kernel(x):
    m, n = x.shape
    h2 = m // 2
    q = m // 4
    e = m // 8

    def body(x_ref, out_ref, recv_a1, recv_b1, recv_a2, recv_b2,
             acc_a, acc_b, send_sems, recv_sems):
        my = lax.axis_index("i")
        cx = (my >> 1) & 1
        cy = (my ^ (my >> 1)) & 1
        py = my ^ 1
        px = 3 - my

        barrier_sem = pltpu.get_barrier_semaphore()
        for nbr in [py, px]:
            pl.semaphore_signal(
                barrier_sem, inc=1,
                device_id=(nbr,), device_id_type=pl.DeviceIdType.MESH,
            )
        pl.semaphore_wait(barrier_sem, 2)

        def exch(idx, src, dst, partner):
            return pltpu.make_async_remote_copy(
                src_ref=src, dst_ref=dst,
                send_sem=send_sems.at[idx], recv_sem=recv_sems.at[idx],
                device_id=(partner,), device_id_type=pl.DeviceIdType.MESH,
            )

        a1 = exch(0, x_ref.at[pl.ds((1 - cy) * q, q)], recv_a1, py)
        a1.start()
        b1 = exch(1, x_ref.at[pl.ds(h2 + (1 - cx) * q, q)], recv_b1, px)
        b1.start()

        a1.wait_recv()
        acc_a[...] = x_ref[pl.ds(cy * q, q), :] + recv_a1[...]
        a2 = exch(2, acc_a.at[pl.ds((1 - cx) * e, e)], recv_a2, px)
        a2.start()

        b1.wait_recv()
        acc_b[...] = x_ref[pl.ds(h2 + cx * q, q), :] + recv_b1[...]
        b2 = exch(3, acc_b.at[pl.ds((1 - cy) * e, e)], recv_b2, py)
        b2.start()

        r_a = cy * q + cx * e
        r_b = h2 + cx * q + cy * e

        a2.wait_recv()
        out_ref[pl.ds(r_a, e), :] = acc_a[pl.ds(cx * e, e), :] + recv_a2[...]
        a3 = exch(4, out_ref.at[pl.ds(r_a, e)], out_ref.at[pl.ds(r_a, e)], px)
        a3.start()

        b2.wait_recv()
        out_ref[pl.ds(r_b, e), :] = acc_b[pl.ds(cy * e, e), :] + recv_b2[...]
        b3 = exch(5, out_ref.at[pl.ds(r_b, e)], out_ref.at[pl.ds(r_b, e)], py)
        b3.start()

        a3.wait_recv()
        a4 = exch(6, out_ref.at[pl.ds(cy * q, q)],
                  out_ref.at[pl.ds(cy * q, q)], py)
        a4.start()

        b3.wait_recv()
        b4 = exch(7, out_ref.at[pl.ds(h2 + cx * q, q)],
                  out_ref.at[pl.ds(h2 + cx * q, q)], px)
        b4.start()

        a4.wait_recv()
        b4.wait_recv()
        for r in [a1, b1, a2, b2, a3, b3, a4, b4]:
            r.wait_send()

    return pl.pallas_call(
        body,
        out_shape=jax.ShapeDtypeStruct((m, n), x.dtype),
        in_specs=[pl.BlockSpec(memory_space=pltpu.VMEM)],
        out_specs=pl.BlockSpec(memory_space=pltpu.VMEM),
        scratch_shapes=[
            pltpu.VMEM((q, n), x.dtype),
            pltpu.VMEM((q, n), x.dtype),
            pltpu.VMEM((e, n), x.dtype),
            pltpu.VMEM((e, n), x.dtype),
            pltpu.VMEM((q, n), x.dtype),
            pltpu.VMEM((q, n), x.dtype),
            pltpu.SemaphoreType.DMA((8,)),
            pltpu.SemaphoreType.DMA((8,)),
        ],
        compiler_params=pltpu.CompilerParams(collective_id=0),
    )(x)


# device time: 26821 ns/iter; 2.9164x vs baseline; 1.0963x over previous
import jax
import jax.numpy as jnp
from jax import lax
from jax.experimental import pallas as pl
from jax.experimental.pallas import tpu as pltpu

N_DEV = 4
N_STREAMS = 4


def kernel(x):
    m, n = x.shape
    sub = m // N_STREAMS
    q = sub // 2
    e = sub // 4

    def body(x_ref, out_ref, *scratch):
        recvs1 = scratch[0:4]
        recvs2 = scratch[4:8]
        accs = scratch[8:12]
        send_sems, recv_sems = scratch[12], scratch[13]

        my = lax.axis_index("i")
        cx = (my >> 1) & 1
        cy = (my ^ (my >> 1)) & 1
        py = my ^ 1
        px = 3 - my

        barrier_sem = pltpu.get_barrier_semaphore()
        for nbr in [py, px]:
            pl.semaphore_signal(
                barrier_sem, inc=1,
                device_id=(nbr,), device_id_type=pl.DeviceIdType.MESH,
            )
        pl.semaphore_wait(barrier_sem, 2)

        def exch(idx, src, dst, partner):
            return pltpu.make_async_remote_copy(
                src_ref=src, dst_ref=dst,
                send_sem=send_sems.at[idx], recv_sem=recv_sems.at[idx],
                device_id=(partner,), device_id_type=pl.DeviceIdType.MESH,
            )

        def geom(s):
            base = s * sub
            if s < 2:
                k1, k2 = cy, cx
                p1, p2 = py, px
            else:
                k1, k2 = cx, cy
                p1, p2 = px, py
            red = base + k1 * q + k2 * e
            return base, k1, k2, p1, p2, red

        G = [geom(s) for s in range(N_STREAMS)]
        sem = lambda s, st: 4 * s + st

        st1 = []
        for s in range(N_STREAMS):
            base, k1, _, p1, _, _ = G[s]
            r = exch(sem(s, 0), x_ref.at[pl.ds(base + (1 - k1) * q, q)],
                     recvs1[s], p1)
            r.start()
            st1.append(r)

        st2 = []
        for s in range(N_STREAMS):
            base, k1, k2, _, p2, _ = G[s]
            st1[s].wait_recv()
            accs[s][...] = x_ref[pl.ds(base + k1 * q, q), :] + recvs1[s][...]
            r = exch(sem(s, 1), accs[s].at[pl.ds((1 - k2) * e, e)],
                     recvs2[s], p2)
            r.start()
            st2.append(r)

        st3 = []
        for s in range(N_STREAMS):
            _, _, k2, _, p2, red = G[s]
            st2[s].wait_recv()
            out_ref[pl.ds(red, e), :] = (
                accs[s][pl.ds(k2 * e, e), :] + recvs2[s][...]
            )
            r = exch(sem(s, 2), out_ref.at[pl.ds(red, e)],
                     out_ref.at[pl.ds(red, e)], p2)
            r.start()
            st3.append(r)

        st4 = []
        for s in range(N_STREAMS):
            base, k1, _, p1, _, _ = G[s]
            st3[s].wait_recv()
            r = exch(sem(s, 3), out_ref.at[pl.ds(base + k1 * q, q)],
                     out_ref.at[pl.ds(base + k1 * q, q)], p1)
            r.start()
            st4.append(r)

        for s in range(N_STREAMS):
            st4[s].wait_recv()
        for r in st1 + st2 + st3 + st4:
            r.wait_send()

    return pl.pallas_call(
        body,
        out_shape=jax.ShapeDtypeStruct((m, n), x.dtype),
        in_specs=[pl.BlockSpec(memory_space=pltpu.VMEM)],
        out_specs=pl.BlockSpec(memory_space=pltpu.VMEM),
        scratch_shapes=(
            [pltpu.VMEM((q, n), x.dtype) for _ in range(N_STREAMS)]
            + [pltpu.VMEM((e, n), x.dtype) for _ in range(N_STREAMS)]
            + [pltpu.VMEM((q, n), x.dtype) for _ in range(N_STREAMS)]
            + [
                pltpu.SemaphoreType.DMA((16,)),
                pltpu.SemaphoreType.DMA((16,)),
            ]
        ),
        compiler_params=pltpu.CompilerParams(collective_id=0),
    )(x)


# device time: 24018 ns/iter; 3.2567x vs baseline; 1.1167x over previous
import jax
import jax.numpy as jnp
from jax import lax
from jax.experimental import pallas as pl
from jax.experimental.pallas import tpu as pltpu

N_DEV = 4
N_STREAMS = 8
ORDER = [0, 1, 2, 3, 4, 5, 6, 7]


def kernel(x):
    m, n = x.shape
    sub = m // N_STREAMS
    q = sub // 2
    e = sub // 4

    def body(x_ref, out_ref, *scratch):
        recvs1 = scratch[0:N_STREAMS]
        recvs2 = scratch[N_STREAMS:2 * N_STREAMS]
        accs = scratch[2 * N_STREAMS:3 * N_STREAMS]
        send_sems, recv_sems = scratch[3 * N_STREAMS], scratch[3 * N_STREAMS + 1]

        my = lax.axis_index("i")
        cx = (my >> 1) & 1
        cy = (my ^ (my >> 1)) & 1
        py = my ^ 1
        px = 3 - my

        barrier_sem = pltpu.get_barrier_semaphore()
        for nbr in [py, px]:
            pl.semaphore_signal(
                barrier_sem, inc=1,
                device_id=(nbr,), device_id_type=pl.DeviceIdType.MESH,
            )
        pl.semaphore_wait(barrier_sem, 2)

        def exch(idx, src, dst, partner):
            return pltpu.make_async_remote_copy(
                src_ref=src, dst_ref=dst,
                send_sem=send_sems.at[idx], recv_sem=recv_sems.at[idx],
                device_id=(partner,), device_id_type=pl.DeviceIdType.MESH,
            )

        def geom(s):
            base = s * sub
            if s % 2 == 0:
                k1, k2 = cy, cx
                p1, p2 = py, px
            else:
                k1, k2 = cx, cy
                p1, p2 = px, py
            keep = base + k1 * q
            red = keep + k2 * e
            return base, k1, k2, p1, p2, keep, red

        G = [geom(s) for s in range(N_STREAMS)]
        sem = lambda s, st: 4 * s + st

        st1 = {}
        for s in ORDER:
            base, k1, _, p1, _, _, _ = G[s]
            r = exch(sem(s, 0), x_ref.at[pl.ds(base + (1 - k1) * q, q)],
                     recvs1[s], p1)
            r.start()
            st1[s] = r

        st2 = {}
        for s in ORDER:
            _, _, k2, _, p2, keep, _ = G[s]
            st1[s].wait_recv()
            off = (1 - k2) * e
            accs[s][...] = (
                x_ref[pl.ds(keep + off, e), :] + recvs1[s][pl.ds(off, e), :]
            )
            r = exch(sem(s, 1), accs[s], recvs2[s], p2)
            r.start()
            st2[s] = r

        st3 = {}
        for s in ORDER:
            _, _, k2, _, p2, keep, red = G[s]
            st2[s].wait_recv()
            off = k2 * e
            out_ref[pl.ds(red, e), :] = (
                x_ref[pl.ds(keep + off, e), :]
                + recvs1[s][pl.ds(off, e), :]
                + recvs2[s][...]
            )
            r = exch(sem(s, 2), out_ref.at[pl.ds(red, e)],
                     out_ref.at[pl.ds(red, e)], p2)
            r.start()
            st3[s] = r

        st4 = {}
        for s in ORDER:
            _, _, _, p1, _, keep, _ = G[s]
            st3[s].wait_recv()
            r = exch(sem(s, 3), out_ref.at[pl.ds(keep, q)],
                     out_ref.at[pl.ds(keep, q)], p1)
            r.start()
            st4[s] = r

        for s in ORDER:
            st4[s].wait_recv()
        for rs in (st1, st2, st3, st4):
            for s in ORDER:
                rs[s].wait_send()

    return pl.pallas_call(
        body,
        out_shape=jax.ShapeDtypeStruct((m, n), x.dtype),
        in_specs=[pl.BlockSpec(memory_space=pltpu.VMEM)],
        out_specs=pl.BlockSpec(memory_space=pltpu.VMEM),
        scratch_shapes=(
            [pltpu.VMEM((q, n), x.dtype) for _ in range(N_STREAMS)]
            + [pltpu.VMEM((e, n), x.dtype) for _ in range(N_STREAMS)]
            + [pltpu.VMEM((e, n), x.dtype) for _ in range(N_STREAMS)]
            + [
                pltpu.SemaphoreType.DMA((4 * N_STREAMS,)),
                pltpu.SemaphoreType.DMA((4 * N_STREAMS,)),
            ]
        ),
        compiler_params=pltpu.CompilerParams(collective_id=0),
    )(x)
